# baseline (speedup 1.0000x reference)
import functools

import jax
import jax.numpy as jnp
from jax import lax
from jax.experimental import pallas as pl
from jax.experimental.pallas import tpu as pltpu
from jax.sharding import Mesh, PartitionSpec as P

_VMEM_LIMIT = 60 * 1024 * 1024
_TILE = 64
_SB = 256


def _layer_kernel(first, h_ref, hloc_ref, adj_ref, wks_ref, bks_ref,
                  wqvT_ref, bqvT_ref, cv_ref, out_ref,
                  u_ref, sk_ref, wt_ref, vt_ref):
    t = pl.program_id(0)
    hidden = wt_ref.shape[0]
    n_nodes = wt_ref.shape[1]

    @pl.when(t == 0)
    def _():
        ks = jnp.dot(hloc_ref[...], wks_ref[...],
                     preferred_element_type=jnp.float32) + bks_ref[...]
        u_ref[...] = ks[:, :hidden]
        sk_ref[...] = ks[:, hidden:]
        qv = jnp.dot(wqvT_ref[...], h_ref[...].T,
                     preferred_element_type=jnp.float32) + bqvT_ref[...]
        wt_ref[...] = qv[:hidden, :]
        vt_ref[...] = qv[hidden:, :]

    l0 = pl.multiple_of(t * _TILE, _TILE)
    u = u_ref[pl.ds(l0, _TILE), :]
    sk = sk_ref[pl.ds(l0, _TILE), :]
    ub = u[:, :, None]

    acc = jnp.zeros((_TILE, hidden), jnp.float32)
    for i in range(n_nodes // _SB):
        s0 = pl.multiple_of(i * _SB, _SB)
        w = wt_ref[:, pl.ds(s0, _SB)]
        v = vt_ref[:, pl.ds(s0, _SB)]
        a = adj_ref[:, pl.ds(s0, _SB)]
        af = a[:, None, :].astype(jnp.float32)
        g = jax.nn.sigmoid(ub + w[None, :, :]) * af
        acc = acc + jnp.sum(g * v[None, :, :], axis=-1)

    row = jnp.maximum(acc + sk, 0.0)
    row = row * cv_ref[0:1, :] + cv_ref[1:2, :]
    if first:
        out_ref[...] = row
    else:
        out_ref[...] = row + hloc_ref[pl.ds(l0, _TILE), :]


def _run_layer(first, h, h_loc, adj_loc, wks, bks, wqvT, bqvT, cv):
    n, hidden = h.shape
    n_loc = adj_loc.shape[0]
    n_tiles = n_loc // _TILE
    body = functools.partial(_layer_kernel, first)
    return pl.pallas_call(
        body,
        out_shape=jax.ShapeDtypeStruct((n_loc, hidden), jnp.float32),
        grid_spec=pltpu.PrefetchScalarGridSpec(
            num_scalar_prefetch=0,
            grid=(n_tiles,),
            in_specs=[
                pl.BlockSpec((n, hidden), lambda t: (0, 0)),
                pl.BlockSpec((n_loc, hidden), lambda t: (0, 0)),
                pl.BlockSpec((_TILE, n), lambda t: (t, 0)),
                pl.BlockSpec((hidden, 2 * hidden), lambda t: (0, 0)),
                pl.BlockSpec((1, 2 * hidden), lambda t: (0, 0)),
                pl.BlockSpec((2 * hidden, hidden), lambda t: (0, 0)),
                pl.BlockSpec((2 * hidden, 1), lambda t: (0, 0)),
                pl.BlockSpec((2, hidden), lambda t: (0, 0)),
            ],
            out_specs=pl.BlockSpec((_TILE, hidden), lambda t: (t, 0)),
            scratch_shapes=[
                pltpu.VMEM((n_loc, hidden), jnp.float32),
                pltpu.VMEM((n_loc, hidden), jnp.float32),
                pltpu.VMEM((hidden, n), jnp.float32),
                pltpu.VMEM((hidden, n), jnp.float32),
            ]),
        compiler_params=pltpu.CompilerParams(
            dimension_semantics=("arbitrary",),
            vmem_limit_bytes=_VMEM_LIMIT),
    )(h, h_loc, adj_loc, wks, bks, wqvT, bqvT, cv)


def _head_call(n_nodes, h, w1, hv, w2, b2):
    return pl.pallas_call(
        functools.partial(_head_kernel, n_nodes),
        out_shape=jax.ShapeDtypeStruct((1, 1), jnp.float32),
        in_specs=[pl.BlockSpec(memory_space=pltpu.MemorySpace.VMEM)] * 5,
        out_specs=pl.BlockSpec(memory_space=pltpu.MemorySpace.VMEM),
        compiler_params=pltpu.CompilerParams(vmem_limit_bytes=_VMEM_LIMIT),
    )(h, w1, hv, w2, b2)


def _head_kernel(n_nodes, h_ref, w1_ref, hv_ref, w2_ref, b2_ref, out_ref):
    g = jnp.sum(h_ref[...], axis=0, keepdims=True) * (1.0 / n_nodes)
    z = jnp.dot(g, w1_ref[...],
                preferred_element_type=jnp.float32) + hv_ref[0:1, :]
    z = jnp.maximum(z, 0.0)
    z = z * hv_ref[1:2, :] + hv_ref[2:3, :]
    out_ref[...] = jnp.dot(z, w2_ref[...],
                           preferred_element_type=jnp.float32) + b2_ref[...]


def _forward(nshards, h, adj_loc, layer_params, w1, hv, w2, b2):
    n = h.shape[0]
    half = n // nshards
    if nshards > 1:
        idx = lax.axis_index("c")
    for l, (wks, bks, wqvT, bqvT, cv) in enumerate(layer_params):
        if nshards > 1:
            h_loc = lax.dynamic_slice_in_dim(h, idx * half, half, 0)
        else:
            h_loc = h
        h_new = _run_layer(l == 0, h, h_loc, adj_loc, wks, bks, wqvT, bqvT, cv)
        if nshards > 1:
            h = lax.all_gather(h_new, "c", axis=0, tiled=True)
        else:
            h = h_new
    return _head_call(n, h, w1, hv, w2, b2)


def kernel(x, adj,
           wk_0, bk_0, wq_0, bq_0, wv_0, bv_0, ws_0, cb_0, bn_scale_0, bn_shift_0,
           wk_1, bk_1, wq_1, bq_1, wv_1, bv_1, ws_1, cb_1, bn_scale_1, bn_shift_1,
           wk_2, bk_2, wq_2, bq_2, wv_2, bv_2, ws_2, cb_2, bn_scale_2, bn_shift_2,
           wk_3, bk_3, wq_3, bq_3, wv_3, bv_3, ws_3, cb_3, bn_scale_3, bn_shift_3,
           wk_4, bk_4, wq_4, bq_4, wv_4, bv_4, ws_4, cb_4, bn_scale_4, bn_shift_4,
           head_w1, head_b1, head_bn_scale, head_bn_shift, head_w2, head_b2):
    layers = [
        (wk_0, bk_0, wq_0, bq_0, wv_0, bv_0, ws_0, cb_0, bn_scale_0, bn_shift_0),
        (wk_1, bk_1, wq_1, bq_1, wv_1, bv_1, ws_1, cb_1, bn_scale_1, bn_shift_1),
        (wk_2, bk_2, wq_2, bq_2, wv_2, bv_2, ws_2, cb_2, bn_scale_2, bn_shift_2),
        (wk_3, bk_3, wq_3, bq_3, wv_3, bv_3, ws_3, cb_3, bn_scale_3, bn_shift_3),
        (wk_4, bk_4, wq_4, bq_4, wv_4, bv_4, ws_4, cb_4, bn_scale_4, bn_shift_4),
    ]
    n = x.shape[0]
    hidden = wk_0.shape[1]
    adj = adj.astype(jnp.bfloat16)
    h0 = jnp.pad(x.astype(jnp.float32), ((0, 0), (0, hidden - x.shape[1])))

    layer_params = []
    for wk, bk, wq, bq, wv, bv, ws, cb, bns, bnsh in layers:
        cin = wk.shape[0]
        if cin < hidden:
            pad = ((0, hidden - cin), (0, 0))
            wk, wq = jnp.pad(wk, pad), jnp.pad(wq, pad)
            wv, ws = jnp.pad(wv, pad), jnp.pad(ws, pad)
        wks = jnp.concatenate([wk, ws], axis=1)
        bks = jnp.concatenate([bk, cb], axis=1)
        wqvT = jnp.concatenate([wq.T, wv.T], axis=0)
        bqvT = jnp.concatenate([bq.T, bv.T], axis=0)
        cv = jnp.concatenate([bns, bnsh], axis=0)
        layer_params.append((wks, bks, wqvT, bqvT, cv))

    hv = jnp.concatenate([head_b1, head_bn_scale, head_bn_shift], axis=0)

    devs = jax.devices()
    nshards = 2 if len(devs) >= 2 else 1
    if nshards > 1:
        mesh = Mesh(devs[:2], ("c",))
        fwd = jax.shard_map(
            functools.partial(_forward, nshards),
            mesh=mesh,
            in_specs=(P(None, None), P("c", None),
                      jax.tree_util.tree_map(lambda _: P(None, None),
                                             layer_params),
                      P(None, None), P(None, None), P(None, None),
                      P(None, None)),
            out_specs=P(None, None),
            check_vma=False,
        )
        out = fwd(h0, adj, layer_params, head_w1, hv, head_w2, head_b2)
    else:
        out = _forward(1, h0, adj, layer_params, head_w1, hv, head_w2, head_b2)
    return out[:, 0]


# speedup vs baseline: 1.2452x; 1.2452x over previous
import functools

import jax
import jax.numpy as jnp
from jax import lax
from jax.experimental import pallas as pl
from jax.experimental.pallas import tpu as pltpu
from jax.sharding import Mesh, PartitionSpec as P

_VMEM_LIMIT = 60 * 1024 * 1024
_TILE = 64
_SB = 256


def _layer_kernel(first, h_ref, hloc_ref, adj_ref, wks_ref, bks_ref,
                  wqvT_ref, bqvT_ref, cv_ref, out_ref,
                  u_ref, sk_ref, wt_ref, vt_ref):
    t = pl.program_id(0)
    hidden = wt_ref.shape[0]
    n_nodes = wt_ref.shape[1]

    @pl.when(t == 0)
    def _():
        ks = jnp.dot(hloc_ref[...], wks_ref[...],
                     preferred_element_type=jnp.float32) + bks_ref[...]
        u_ref[...] = ks[:, :hidden].astype(jnp.bfloat16)
        sk_ref[...] = ks[:, hidden:]
        qv = jnp.dot(wqvT_ref[...], h_ref[...].T,
                     preferred_element_type=jnp.float32) + bqvT_ref[...]
        wt_ref[...] = qv[:hidden, :].astype(jnp.bfloat16)
        vt_ref[...] = qv[hidden:, :]

    l0 = pl.multiple_of(t * _TILE, _TILE)
    u = u_ref[pl.ds(l0, _TILE), :]
    sk = sk_ref[pl.ds(l0, _TILE), :]
    ub = u[:, :, None]

    acc = jnp.zeros((_TILE, hidden), jnp.float32)
    for i in range(n_nodes // _SB):
        s0 = pl.multiple_of(i * _SB, _SB)
        w = wt_ref[:, pl.ds(s0, _SB)]
        v = vt_ref[:, pl.ds(s0, _SB)]
        a = adj_ref[:, pl.ds(s0, _SB)]
        one = jnp.bfloat16(1.0)
        g = (jnp.tanh(ub + w[None, :, :]) + one) * a[:, None, :]
        acc = acc + jnp.sum(g.astype(jnp.float32) * v[None, :, :], axis=-1)

    row = jnp.maximum(acc + sk, 0.0)
    row = row * cv_ref[0:1, :] + cv_ref[1:2, :]
    if first:
        out_ref[...] = row
    else:
        out_ref[...] = row + hloc_ref[pl.ds(l0, _TILE), :]


def _run_layer(first, h, h_loc, adj_loc, wks, bks, wqvT, bqvT, cv):
    n, hidden = h.shape
    n_loc = adj_loc.shape[0]
    n_tiles = n_loc // _TILE
    body = functools.partial(_layer_kernel, first)
    return pl.pallas_call(
        body,
        out_shape=jax.ShapeDtypeStruct((n_loc, hidden), jnp.float32),
        grid_spec=pltpu.PrefetchScalarGridSpec(
            num_scalar_prefetch=0,
            grid=(n_tiles,),
            in_specs=[
                pl.BlockSpec((n, hidden), lambda t: (0, 0)),
                pl.BlockSpec((n_loc, hidden), lambda t: (0, 0)),
                pl.BlockSpec((_TILE, n), lambda t: (t, 0)),
                pl.BlockSpec((hidden, 2 * hidden), lambda t: (0, 0)),
                pl.BlockSpec((1, 2 * hidden), lambda t: (0, 0)),
                pl.BlockSpec((2 * hidden, hidden), lambda t: (0, 0)),
                pl.BlockSpec((2 * hidden, 1), lambda t: (0, 0)),
                pl.BlockSpec((2, hidden), lambda t: (0, 0)),
            ],
            out_specs=pl.BlockSpec((_TILE, hidden), lambda t: (t, 0)),
            scratch_shapes=[
                pltpu.VMEM((n_loc, hidden), jnp.bfloat16),
                pltpu.VMEM((n_loc, hidden), jnp.float32),
                pltpu.VMEM((hidden, n), jnp.bfloat16),
                pltpu.VMEM((hidden, n), jnp.float32),
            ]),
        compiler_params=pltpu.CompilerParams(
            dimension_semantics=("arbitrary",),
            vmem_limit_bytes=_VMEM_LIMIT),
    )(h, h_loc, adj_loc, wks, bks, wqvT, bqvT, cv)


def _head_call(n_nodes, h, w1, hv, w2, b2):
    return pl.pallas_call(
        functools.partial(_head_kernel, n_nodes),
        out_shape=jax.ShapeDtypeStruct((1, 1), jnp.float32),
        in_specs=[pl.BlockSpec(memory_space=pltpu.MemorySpace.VMEM)] * 5,
        out_specs=pl.BlockSpec(memory_space=pltpu.MemorySpace.VMEM),
        compiler_params=pltpu.CompilerParams(vmem_limit_bytes=_VMEM_LIMIT),
    )(h, w1, hv, w2, b2)


def _head_kernel(n_nodes, h_ref, w1_ref, hv_ref, w2_ref, b2_ref, out_ref):
    g = jnp.sum(h_ref[...], axis=0, keepdims=True) * (1.0 / n_nodes)
    z = jnp.dot(g, w1_ref[...],
                preferred_element_type=jnp.float32) + hv_ref[0:1, :]
    z = jnp.maximum(z, 0.0)
    z = z * hv_ref[1:2, :] + hv_ref[2:3, :]
    out_ref[...] = jnp.dot(z, w2_ref[...],
                           preferred_element_type=jnp.float32) + b2_ref[...]


def _forward(nshards, h, adj_loc, layer_params, w1, hv, w2, b2):
    n = h.shape[0]
    half = n // nshards
    if nshards > 1:
        idx = lax.axis_index("c")
    for l, (wks, bks, wqvT, bqvT, cv) in enumerate(layer_params):
        if nshards > 1:
            h_loc = lax.dynamic_slice_in_dim(h, idx * half, half, 0)
        else:
            h_loc = h
        h_new = _run_layer(l == 0, h, h_loc, adj_loc, wks, bks, wqvT, bqvT, cv)
        if nshards > 1:
            h = lax.all_gather(h_new, "c", axis=0, tiled=True)
        else:
            h = h_new
    return _head_call(n, h, w1, hv, w2, b2)


def kernel(x, adj,
           wk_0, bk_0, wq_0, bq_0, wv_0, bv_0, ws_0, cb_0, bn_scale_0, bn_shift_0,
           wk_1, bk_1, wq_1, bq_1, wv_1, bv_1, ws_1, cb_1, bn_scale_1, bn_shift_1,
           wk_2, bk_2, wq_2, bq_2, wv_2, bv_2, ws_2, cb_2, bn_scale_2, bn_shift_2,
           wk_3, bk_3, wq_3, bq_3, wv_3, bv_3, ws_3, cb_3, bn_scale_3, bn_shift_3,
           wk_4, bk_4, wq_4, bq_4, wv_4, bv_4, ws_4, cb_4, bn_scale_4, bn_shift_4,
           head_w1, head_b1, head_bn_scale, head_bn_shift, head_w2, head_b2):
    layers = [
        (wk_0, bk_0, wq_0, bq_0, wv_0, bv_0, ws_0, cb_0, bn_scale_0, bn_shift_0),
        (wk_1, bk_1, wq_1, bq_1, wv_1, bv_1, ws_1, cb_1, bn_scale_1, bn_shift_1),
        (wk_2, bk_2, wq_2, bq_2, wv_2, bv_2, ws_2, cb_2, bn_scale_2, bn_shift_2),
        (wk_3, bk_3, wq_3, bq_3, wv_3, bv_3, ws_3, cb_3, bn_scale_3, bn_shift_3),
        (wk_4, bk_4, wq_4, bq_4, wv_4, bv_4, ws_4, cb_4, bn_scale_4, bn_shift_4),
    ]
    n = x.shape[0]
    hidden = wk_0.shape[1]
    adj = adj.astype(jnp.bfloat16)
    h0 = jnp.pad(x.astype(jnp.float32), ((0, 0), (0, hidden - x.shape[1])))

    layer_params = []
    for wk, bk, wq, bq, wv, bv, ws, cb, bns, bnsh in layers:
        cin = wk.shape[0]
        if cin < hidden:
            pad = ((0, hidden - cin), (0, 0))
            wk, wq = jnp.pad(wk, pad), jnp.pad(wq, pad)
            wv, ws = jnp.pad(wv, pad), jnp.pad(ws, pad)
        wks = jnp.concatenate([0.5 * wk, ws], axis=1)
        bks = jnp.concatenate([0.5 * bk, cb], axis=1)
        wqvT = jnp.concatenate([0.5 * wq.T, 0.5 * wv.T], axis=0)
        bqvT = jnp.concatenate([0.5 * bq.T, 0.5 * bv.T], axis=0)
        cv = jnp.concatenate([bns, bnsh], axis=0)
        layer_params.append((wks, bks, wqvT, bqvT, cv))

    hv = jnp.concatenate([head_b1, head_bn_scale, head_bn_shift], axis=0)

    devs = jax.devices()
    nshards = 2 if len(devs) >= 2 else 1
    if nshards > 1:
        mesh = Mesh(devs[:2], ("c",))
        fwd = jax.shard_map(
            functools.partial(_forward, nshards),
            mesh=mesh,
            in_specs=(P(None, None), P("c", None),
                      jax.tree_util.tree_map(lambda _: P(None, None),
                                             layer_params),
                      P(None, None), P(None, None), P(None, None),
                      P(None, None)),
            out_specs=P(None, None),
            check_vma=False,
        )
        out = fwd(h0, adj, layer_params, head_w1, hv, head_w2, head_b2)
    else:
        out = _forward(1, h0, adj, layer_params, head_w1, hv, head_w2, head_b2)
    return out[:, 0]
